# Initial kernel scaffold; baseline (speedup 1.0000x reference)
import jax
import jax.numpy as jnp
from jax.experimental import pallas as pl


def kernel(inputs):
    raise NotImplementedError("write your pallas kernel here")


# speedup vs baseline: 3.5437x; 3.5437x over previous
import jax
import jax.numpy as jnp
import numpy as np
from jax.experimental import pallas as pl

_NB_SYMB = 14
_FFT = 4096
_SPACING = 4
_NPIL = _FFT // _SPACING
_BATCH = 256
_BBLK = 64
_KCHUNK = 512
_NK = _FFT // _KCHUNK
_XW = 256


def _freq_interp_blocks() -> np.ndarray:
    e = np.zeros((_NPIL, _FFT), np.float32)
    for k in range(_FFT):
        q = k // _SPACING
        if q >= _NPIL - 1:
            e[_NPIL - 1, k] = 1.0
        else:
            w = (k % _SPACING) / _SPACING
            e[q, k] = 1.0 - w
            e[q + 1, k] = w
    blocks = np.zeros((_NK, _XW, _KCHUNK), np.float32)
    for j in range(_NK):
        s = min(j * (_KCHUNK // _SPACING), _NPIL - _XW)
        blocks[j] = e[s:s + _XW, j * _KCHUNK:(j + 1) * _KCHUNK]
    return blocks


_E2 = _freq_interp_blocks()
_TNORM = np.clip((np.arange(_NB_SYMB) - 2.0) / 9.0, 0.0, 1.0).astype(np.float32)
_QPC = _KCHUNK // _SPACING


def _body(x_ref, e_ref, o_ref):
    j = pl.program_id(1)
    start = jnp.minimum(j * _QPC, _NPIL - _XW)
    ej = e_ref[j]
    x0 = x_ref[:, pl.ds(start, _XW)]
    x1 = x_ref[:, pl.ds(_NPIL + start, _XW)]
    hf0 = jax.lax.dot(
        x0, ej,
        precision=jax.lax.Precision.HIGHEST,
        preferred_element_type=jnp.float32,
    )
    hf1 = jax.lax.dot(
        x1, ej,
        precision=jax.lax.Precision.HIGHEST,
        preferred_element_type=jnp.float32,
    )
    d = hf1 - hf0
    for s in range(_NB_SYMB):
        t = float(_TNORM[s])
        if t == 0.0:
            o_ref[:, s, :] = hf0
        elif t == 1.0:
            o_ref[:, s, :] = hf1
        else:
            o_ref[:, s, :] = hf0 + t * d


@jax.jit
def kernel(inputs):
    b = inputs.shape[0]
    e2 = jnp.asarray(_E2)
    out = pl.pallas_call(
        _body,
        grid=(b // _BBLK, _NK),
        in_specs=[
            pl.BlockSpec((_BBLK, 2 * _NPIL), lambda i, j: (i, 0)),
            pl.BlockSpec((_NK, _XW, _KCHUNK), lambda i, j: (0, 0, 0)),
        ],
        out_specs=pl.BlockSpec((_BBLK, _NB_SYMB, _KCHUNK), lambda i, j: (i, 0, j)),
        out_shape=jax.ShapeDtypeStruct((b, _NB_SYMB, _FFT), inputs.dtype),
    )(inputs, e2)
    return out
